# baseline (speedup 1.0000x reference)
import jax
import jax.numpy as jnp
from jax import lax
from jax.experimental import pallas as pl

_MULT_SIZE = 1.2
_NBLK = 8
_BLK = 5120


def _pad_body(species_ref, natoms_ref, batch_ref, coordsT_ref,
              species_out_ref, natoms_out_ref, batch_out_ref, coordsT_out_ref,
              true_atoms_ref, *, nat, nsys, pad_nat):
    i = pl.program_id(0)
    add = pad_nat - nat
    idx = i * _BLK + lax.broadcasted_iota(jnp.int32, (_BLK,), 0)
    in_range = idx < nat

    s = jnp.where(in_range, species_ref[...], -1)
    species_out_ref[...] = s
    true_atoms_ref[...] = s > 0
    batch_out_ref[...] = jnp.where(in_range, batch_ref[...], nsys)
    coordsT_out_ref[...] = jnp.where(in_range[None, :], coordsT_ref[...], 0.0)

    @pl.when(i == 0)
    def _():
        natoms_out_ref[0:nsys] = natoms_ref[...]
        natoms_out_ref[nsys:nsys + 1] = jnp.full((1,), add, natoms_ref.dtype)


def kernel(species, natoms, batch_index, coordinates, cells):
    nat = species.shape[0]
    nsys = natoms.shape[0]
    pad_nat = int(_MULT_SIZE * nat) + 1
    ndim = coordinates.shape[1]
    n_in_blk = (nat + _BLK - 1) // _BLK - 1

    import functools
    body = functools.partial(_pad_body, nat=nat, nsys=nsys, pad_nat=pad_nat)

    out_shape = (
        jax.ShapeDtypeStruct((pad_nat,), species.dtype),
        jax.ShapeDtypeStruct((nsys + 1,), natoms.dtype),
        jax.ShapeDtypeStruct((pad_nat,), batch_index.dtype),
        jax.ShapeDtypeStruct((ndim, pad_nat), coordinates.dtype),
        jax.ShapeDtypeStruct((pad_nat,), jnp.bool_),
    )
    clamp = lambda i: (min(i, n_in_blk) if isinstance(i, int) else
                       jnp.minimum(i, n_in_blk),)
    in_specs = [
        pl.BlockSpec((_BLK,), clamp),
        pl.BlockSpec((nsys,), lambda i: (0,)),
        pl.BlockSpec((_BLK,), clamp),
        pl.BlockSpec((ndim, _BLK), lambda i: (0, jnp.minimum(i, n_in_blk))),
    ]
    out_specs = [
        pl.BlockSpec((_BLK,), lambda i: (i,)),
        pl.BlockSpec((nsys + 1,), lambda i: (0,)),
        pl.BlockSpec((_BLK,), lambda i: (i,)),
        pl.BlockSpec((ndim, _BLK), lambda i: (0, i)),
        pl.BlockSpec((_BLK,), lambda i: (i,)),
    ]
    (species_out, natoms_out, batch_out, coordsT_out,
     true_atoms) = pl.pallas_call(
        body, grid=(_NBLK,), in_specs=in_specs, out_specs=out_specs,
        out_shape=out_shape)(
        species, natoms, batch_index, coordinates.T)

    cells_out = jnp.concatenate(
        [cells, jnp.eye(cells.shape[1], dtype=cells.dtype)[None, :, :]], axis=0)
    true_sys = jnp.arange(nsys + 1) < nsys
    return (species_out, natoms_out, batch_out, coordsT_out.T, cells_out,
            true_atoms, true_sys)
